# baseline (speedup 1.0000x reference)
import functools

import jax
import jax.numpy as jnp
from jax import lax
from jax.experimental import pallas as pl
from jax.experimental.pallas import tpu as pltpu
from jax.experimental.pallas import tpu_sc as plsc

N = 10000
E = 320000
H = 128

NW = 32
EPW = E // NW
CH = 125
NCH = EPW // CH
SCH = 16
NSC = NCH // SCH
DBIG = 640
DLAST = N - 15 * DBIG
NPAD = 16 * DBIG
PR = NPAD // 128

_SC_PARAMS = pltpu.CompilerParams(use_tc_tiling_on_sc=False)
_SC_PARAMS_GATHER = pltpu.CompilerParams(use_tc_tiling_on_sc=False,
                                         needs_layout_passes=False)


def _mesh():
    return plsc.VectorSubcoreMesh(core_axis_name="c", subcore_axis_name="s")


def _seg_sum(h, e3, with_deg: bool):
    out_type = [jax.ShapeDtypeStruct((2, N, H), jnp.float32)]
    if with_deg:
        out_type.append(jax.ShapeDtypeStruct((2, 1, NPAD), jnp.float32))
    scratch = [
        pltpu.VMEM_SHARED((N, H), jnp.float32),
        pltpu.VMEM((32, H), jnp.float32),
        pltpu.VMEM((2, SCH, CH), jnp.int32),
        pltpu.VMEM((2, SCH, CH), jnp.int32),
        pltpu.VMEM((CH, H), jnp.float32),
        pltpu.VMEM((CH, H), jnp.float32),
        pltpu.SemaphoreType.DMA,
        pltpu.SemaphoreType.DMA,
        pltpu.SemaphoreType.DMA,
        pltpu.SemaphoreType.DMA,
        pltpu.SemaphoreType.DMA,
    ]
    if with_deg:
        scratch += [
            pltpu.VMEM_SHARED((NPAD,), jnp.float32),
            pltpu.VMEM((DBIG,), jnp.float32),
            pltpu.VMEM((128,), jnp.float32),
        ]

    def body(h_hbm, e3_hbm, *rest):
        if with_deg:
            (agg_out, deg_out, acc, zbuf, sidx, didx, rows0, rows1,
             gsemA, gsemB, isem0, isem1, zsem, accd, zdeg, ones) = rest
        else:
            (agg_out, acc, zbuf, sidx, didx, rows0, rows1,
             gsemA, gsemB, isem0, isem1, zsem) = rest
        c = lax.axis_index("c")
        s = lax.axis_index("s")
        w = c * 16 + s
        isems = [isem0, isem1]

        def idx_start(g, gi):
            pltpu.async_copy(e3_hbm.at[0, w, pl.ds(g, SCH)], sidx.at[gi],
                             isems[gi])
            pltpu.async_copy(e3_hbm.at[1, w, pl.ds(g, SCH)], didx.at[gi],
                             isems[gi])

        def idx_wait(g, gi):
            pltpu.make_async_copy(e3_hbm.at[0, w, pl.ds(g, SCH)],
                                  sidx.at[gi], isems[gi]).wait()
            pltpu.make_async_copy(e3_hbm.at[1, w, pl.ds(g, SCH)],
                                  didx.at[gi], isems[gi]).wait()

        idx_start(0, 0)

        z16 = jnp.zeros((16,), jnp.float32)

        @pl.loop(0, 32)
        def _(i):
            for j in range(H // 16):
                zbuf[i, pl.ds(j * 16, 16)] = z16

        if with_deg:
            @pl.loop(0, DBIG // 16)
            def _(i):
                zdeg[pl.ds(i * 16, 16)] = z16
            o16 = jnp.ones((16,), jnp.float32)
            for i in range(128 // 16):
                ones[pl.ds(i * 16, 16)] = o16

        for k in range(DBIG // 32):
            @pl.when(s * DBIG + k * 32 < N)
            def _():
                pltpu.async_copy(zbuf, acc.at[pl.ds(s * DBIG + k * 32, 32)],
                                 zsem)
        for k in range(DBIG // 32):
            @pl.when(s * DBIG + k * 32 < N)
            def _():
                pltpu.make_async_copy(
                    zbuf, acc.at[pl.ds(s * DBIG + k * 32, 32)], zsem).wait()
        if with_deg:
            pltpu.sync_copy(zdeg, accd.at[pl.ds(s * DBIG, DBIG)])
        plsc.subcore_barrier()

        def chunk(j, sb, rows, sem, db):
            pltpu.make_async_copy(h_hbm.at[sb.at[j]], rows, sem).wait()
            pltpu.sync_copy(rows, acc.at[db.at[j]], add=True)
            if with_deg:
                pltpu.sync_copy(ones.at[pl.ds(0, CH)],
                                accd.at[db.at[j]], add=True)

        for gi, g in enumerate(range(0, NCH, SCH)):
            b = gi % 2
            sb = sidx.at[b]
            db = didx.at[b]
            idx_wait(g, b)
            if g + SCH < NCH:
                idx_start(g + SCH, 1 - b)
            pltpu.async_copy(h_hbm.at[sb.at[0]], rows0, gsemA)

            @pl.loop(0, SCH, step=2)
            def _(j):
                pltpu.async_copy(h_hbm.at[sb.at[j + 1]], rows1, gsemB)
                chunk(j, sb, rows0, gsemA, db)

                @pl.when(j + 2 < SCH)
                def _():
                    pltpu.async_copy(h_hbm.at[sb.at[j + 2]], rows0, gsemA)
                chunk(j + 1, sb, rows1, gsemB, db)

        plsc.subcore_barrier()

        @pl.when(s < 15)
        def _():
            pltpu.sync_copy(acc.at[pl.ds(s * DBIG, DBIG)],
                            agg_out.at[c, pl.ds(s * DBIG, DBIG)])

        @pl.when(s == 15)
        def _():
            pltpu.sync_copy(acc.at[pl.ds(15 * DBIG, DLAST)],
                            agg_out.at[c, pl.ds(15 * DBIG, DLAST)])
        if with_deg:
            pltpu.sync_copy(accd.at[pl.ds(s * DBIG, DBIG)],
                            deg_out.at[c, 0, pl.ds(s * DBIG, DBIG)])

    fn = pl.kernel(body, out_type=tuple(out_type), mesh=_mesh(),
                   scratch_types=scratch, compiler_params=_SC_PARAMS)
    return fn(h, e3)


def _seg_sum1(p, e3):
    scratch = [
        pltpu.VMEM_SHARED((NPAD,), jnp.float32),
        pltpu.VMEM((NPAD,), jnp.float32),
        pltpu.VMEM((SCH, CH), jnp.int32),
        pltpu.VMEM((SCH, CH), jnp.int32),
        pltpu.VMEM((DBIG,), jnp.float32),
        pltpu.VMEM((CH,), jnp.float32),
        pltpu.VMEM((CH,), jnp.float32),
        pltpu.SemaphoreType.DMA,
        pltpu.SemaphoreType.DMA,
    ]

    def body(p_hbm, e3_hbm, out, accd, table, sidx, didx,
             zdeg, valbuf0, valbuf1, sem0, sem1):
        c = lax.axis_index("c")
        s = lax.axis_index("s")
        w = c * 16 + s

        z16 = jnp.zeros((16,), jnp.float32)

        @pl.loop(0, DBIG // 16)
        def _(i):
            zdeg[pl.ds(i * 16, 16)] = z16
        pltpu.sync_copy(zdeg, accd.at[pl.ds(s * DBIG, DBIG)])
        pltpu.sync_copy(p_hbm, table)
        plsc.subcore_barrier()

        def fill(j, vb):
            for q in range(CH // 16):
                iv = sidx[j, pl.ds(q * 16, 16)]
                vb[pl.ds(q * 16, 16)] = plsc.load_gather(table, [iv])
            iv = sidx[j, pl.ds(CH - 16, 16)]
            vb[pl.ds(CH - 16, 16)] = plsc.load_gather(table, [iv])

        def sstart(j, vb, sem):
            pltpu.async_copy(vb, accd.at[didx.at[j]], sem, add=True)

        def swait(j, vb, sem):
            pltpu.make_async_copy(vb, accd.at[didx.at[j]], sem).wait()

        for g in range(0, NCH, SCH):
            pltpu.sync_copy(e3_hbm.at[0, w, pl.ds(g, SCH)], sidx)
            pltpu.sync_copy(e3_hbm.at[1, w, pl.ds(g, SCH)], didx)
            fill(0, valbuf0)
            sstart(0, valbuf0, sem0)

            @pl.loop(1, SCH - 1, step=2)
            def _(j):
                fill(j, valbuf1)
                sstart(j, valbuf1, sem1)
                swait(j - 1, valbuf0, sem0)
                fill(j + 1, valbuf0)
                sstart(j + 1, valbuf0, sem0)
                swait(j, valbuf1, sem1)

            fill(SCH - 1, valbuf1)
            sstart(SCH - 1, valbuf1, sem1)
            swait(SCH - 2, valbuf0, sem0)
            swait(SCH - 1, valbuf1, sem1)

        plsc.subcore_barrier()
        pltpu.sync_copy(accd.at[pl.ds(s * DBIG, DBIG)],
                        out.at[c, 0, pl.ds(s * DBIG, DBIG)])

    fn = pl.kernel(body,
                   out_type=jax.ShapeDtypeStruct((2, 1, NPAD), jnp.float32),
                   mesh=_mesh(), scratch_types=scratch,
                   compiler_params=_SC_PARAMS_GATHER)
    return fn(p, e3)


_RB1 = 2048
_PB = _RB1 // 128


def _mm(a, b):
    return jnp.dot(a, b, preferred_element_type=jnp.float32)


def _tc_layer0_body(aA, aB, dA, dB, h_, wn, ws, bb, o):
    invd = 1.0 / jnp.maximum(dA[...] + dB[...], 1.0)
    agg = (aA[0] + aB[0]) * invd
    out = _mm(agg, wn[...]) + _mm(h_[...], ws[...]) + bb[...]
    o[...] = jnp.maximum(out, 0.0)


def _tc_layer0(aggp, degA, degB, h, Wn, Ws, b2d):
    row = lambda i: (i, 0)
    fixed = lambda i: (0, 0)
    return pl.pallas_call(
        _tc_layer0_body,
        grid=(5,),
        in_specs=[
            pl.BlockSpec((1, _RB1, H), lambda i: (0, i, 0)),
            pl.BlockSpec((1, _RB1, H), lambda i: (1, i, 0)),
            pl.BlockSpec((_RB1, 1), row),
            pl.BlockSpec((_RB1, 1), row),
            pl.BlockSpec((_RB1, H), row),
            pl.BlockSpec((H, H), fixed),
            pl.BlockSpec((H, H), fixed),
            pl.BlockSpec((1, H), fixed),
        ],
        out_specs=pl.BlockSpec((_RB1, H), row),
        out_shape=jax.ShapeDtypeStruct((N, H), jnp.float32),
    )(aggp, aggp, degA, degB, h, Wn, Ws, b2d)


def _tc_layer1_body(aA, aB, dA, dB, h_, wn, ws, bb, wn2, ws2, b2, op, os):
    invd = 1.0 / jnp.maximum(dA[...] + dB[...], 1.0)
    agg = (aA[0] + aB[0]) * invd
    h2 = _mm(agg, wn[...]) + _mm(h_[...], ws[...]) + bb[...]
    h2 = jnp.maximum(h2, 0.0)
    op[...] = jnp.sum(h2 * wn2[...], axis=1).reshape(_PB, 128)
    os[...] = jnp.sum(h2 * ws2[...], axis=1).reshape(_PB, 128) + b2[...]


def _tc_layer1(aggp, degA, degB, h, Wn, Ws, b2d, wn2, ws2, b11):
    row = lambda i: (i, 0)
    fixed = lambda i: (0, 0)
    return pl.pallas_call(
        _tc_layer1_body,
        grid=(5,),
        in_specs=[
            pl.BlockSpec((1, _RB1, H), lambda i: (0, i, 0)),
            pl.BlockSpec((1, _RB1, H), lambda i: (1, i, 0)),
            pl.BlockSpec((_RB1, 1), row),
            pl.BlockSpec((_RB1, 1), row),
            pl.BlockSpec((_RB1, H), row),
            pl.BlockSpec((H, H), fixed),
            pl.BlockSpec((H, H), fixed),
            pl.BlockSpec((1, H), fixed),
            pl.BlockSpec((1, H), fixed),
            pl.BlockSpec((1, H), fixed),
            pl.BlockSpec((1, 1), fixed),
        ],
        out_specs=[
            pl.BlockSpec((_PB, 128), row),
            pl.BlockSpec((_PB, 128), row),
        ],
        out_shape=[
            jax.ShapeDtypeStruct((PR, 128), jnp.float32),
            jax.ShapeDtypeStruct((PR, 128), jnp.float32),
        ],
    )(aggp, aggp, degA, degB, h, Wn, Ws, b2d, wn2, ws2, b11)


def _tc_combine_body(aa, dd, s2, o):
    d = dd[0] + dd[1]
    invd = 1.0 / jnp.maximum(d, 1.0)
    o[...] = (aa[0] + aa[1]) * invd + s2[...]


def _tc_combine(a2pk, degpk, s2):
    return pl.pallas_call(
        _tc_combine_body,
        grid=(5,),
        in_specs=[
            pl.BlockSpec((2, PR // 5, 128), lambda i: (0, i, 0)),
            pl.BlockSpec((2, PR // 5, 128), lambda i: (0, i, 0)),
            pl.BlockSpec((PR // 5, 128), lambda i: (i, 0)),
        ],
        out_specs=pl.BlockSpec((PR // 5, 128), lambda i: (i, 0)),
        out_shape=jax.ShapeDtypeStruct((PR, 128), jnp.float32),
    )(a2pk, degpk, s2)


def kernel(x, edge_index, Wn0, Ws0, b0, Wn1, Ws1, b1, Wn2, Ws2, b2):
    e3 = edge_index.reshape(2, NW, NCH, CH)

    agg0, degp = _seg_sum(x, e3, with_deg=True)
    degpk = degp.reshape(2, PR, 128)
    degA = degp[0, 0, :N].reshape(N, 1)
    degB = degp[1, 0, :N].reshape(N, 1)

    h1 = _tc_layer0(agg0, degA, degB, x, Wn0, Ws0, b0.reshape(1, H))
    agg1, _ = _seg_sum(h1, e3, with_deg=True)
    p2, s2 = _tc_layer1(agg1, degA, degB, h1, Wn1, Ws1,
                        b1.reshape(1, H), Wn2.reshape(1, H),
                        Ws2.reshape(1, H), b2.reshape(1, 1))
    agg2p = _seg_sum1(p2.reshape(NPAD), e3)
    out = _tc_combine(agg2p.reshape(2, PR, 128), degpk, s2)
    return out.reshape(NPAD)[:N]


# speedup vs baseline: 1.0145x; 1.0145x over previous
import functools

import jax
import jax.numpy as jnp
from jax import lax
from jax.experimental import pallas as pl
from jax.experimental.pallas import tpu as pltpu
from jax.experimental.pallas import tpu_sc as plsc

N = 10000
E = 320000
H = 128

NW = 32
EPW = E // NW
CH = 125
NCH = EPW // CH
SCH = 16
NSC = NCH // SCH
DBIG = 640
DLAST = N - 15 * DBIG
NPAD = 16 * DBIG
PR = NPAD // 128

_SC_PARAMS = pltpu.CompilerParams(use_tc_tiling_on_sc=False)
_SC_PARAMS_GATHER = pltpu.CompilerParams(use_tc_tiling_on_sc=False,
                                         needs_layout_passes=False)


def _mesh():
    return plsc.VectorSubcoreMesh(core_axis_name="c", subcore_axis_name="s")


def _seg_sum(h, e3, with_deg: bool):
    out_type = [jax.ShapeDtypeStruct((2, N, H), jnp.float32)]
    if with_deg:
        out_type.append(jax.ShapeDtypeStruct((2, 1, NPAD), jnp.float32))
    scratch = [
        pltpu.VMEM_SHARED((N, H), jnp.float32),
        pltpu.VMEM((32, H), jnp.float32),
        pltpu.VMEM((2, SCH, CH), jnp.int32),
        pltpu.VMEM((2, SCH, CH), jnp.int32),
        pltpu.VMEM((CH, H), jnp.float32),
        pltpu.VMEM((CH, H), jnp.float32),
        pltpu.SemaphoreType.DMA,
        pltpu.SemaphoreType.DMA,
        pltpu.SemaphoreType.DMA,
        pltpu.SemaphoreType.DMA,
        pltpu.SemaphoreType.DMA,
    ]
    if with_deg:
        scratch += [
            pltpu.VMEM_SHARED((NPAD,), jnp.float32),
            pltpu.VMEM((DBIG,), jnp.float32),
            pltpu.VMEM((128,), jnp.float32),
        ]

    def body(h_hbm, e3_hbm, *rest):
        if with_deg:
            (agg_out, deg_out, acc, zbuf, sidx, didx, rows0, rows1,
             gsemA, gsemB, isem0, isem1, zsem, accd, zdeg, ones) = rest
        else:
            (agg_out, acc, zbuf, sidx, didx, rows0, rows1,
             gsemA, gsemB, isem0, isem1, zsem) = rest
        c = lax.axis_index("c")
        s = lax.axis_index("s")
        w = c * 16 + s
        isems = [isem0, isem1]

        def idx_start(g, gi):
            pltpu.async_copy(e3_hbm.at[0, w, pl.ds(g, SCH)], sidx.at[gi],
                             isems[gi])
            pltpu.async_copy(e3_hbm.at[1, w, pl.ds(g, SCH)], didx.at[gi],
                             isems[gi])

        def idx_wait(g, gi):
            pltpu.make_async_copy(e3_hbm.at[0, w, pl.ds(g, SCH)],
                                  sidx.at[gi], isems[gi]).wait()
            pltpu.make_async_copy(e3_hbm.at[1, w, pl.ds(g, SCH)],
                                  didx.at[gi], isems[gi]).wait()

        idx_start(0, 0)

        z16 = jnp.zeros((16,), jnp.float32)

        @pl.loop(0, 32)
        def _(i):
            for j in range(H // 16):
                zbuf[i, pl.ds(j * 16, 16)] = z16

        if with_deg:
            @pl.loop(0, DBIG // 16)
            def _(i):
                zdeg[pl.ds(i * 16, 16)] = z16
            o16 = jnp.ones((16,), jnp.float32)
            for i in range(128 // 16):
                ones[pl.ds(i * 16, 16)] = o16

        for k in range(DBIG // 32):
            @pl.when(s * DBIG + k * 32 < N)
            def _():
                pltpu.async_copy(zbuf, acc.at[pl.ds(s * DBIG + k * 32, 32)],
                                 zsem)
        for k in range(DBIG // 32):
            @pl.when(s * DBIG + k * 32 < N)
            def _():
                pltpu.make_async_copy(
                    zbuf, acc.at[pl.ds(s * DBIG + k * 32, 32)], zsem).wait()
        if with_deg:
            pltpu.sync_copy(zdeg, accd.at[pl.ds(s * DBIG, DBIG)])
        plsc.subcore_barrier()

        def chunk(j, sb, rows, sem, db):
            pltpu.make_async_copy(h_hbm.at[sb.at[j]], rows, sem).wait()
            pltpu.sync_copy(rows, acc.at[db.at[j]], add=True)
            if with_deg:
                pltpu.sync_copy(ones.at[pl.ds(0, CH)],
                                accd.at[db.at[j]], add=True)

        for gi, g in enumerate(range(0, NCH, SCH)):
            b = gi % 2
            sb = sidx.at[b]
            db = didx.at[b]
            idx_wait(g, b)
            if g + SCH < NCH:
                idx_start(g + SCH, 1 - b)
            pltpu.async_copy(h_hbm.at[sb.at[0]], rows0, gsemA)

            @pl.loop(0, SCH, step=2)
            def _(j):
                pltpu.async_copy(h_hbm.at[sb.at[j + 1]], rows1, gsemB)
                chunk(j, sb, rows0, gsemA, db)

                @pl.when(j + 2 < SCH)
                def _():
                    pltpu.async_copy(h_hbm.at[sb.at[j + 2]], rows0, gsemA)
                chunk(j + 1, sb, rows1, gsemB, db)

        plsc.subcore_barrier()

        @pl.when(s < 15)
        def _():
            pltpu.sync_copy(acc.at[pl.ds(s * DBIG, DBIG)],
                            agg_out.at[c, pl.ds(s * DBIG, DBIG)])

        @pl.when(s == 15)
        def _():
            pltpu.sync_copy(acc.at[pl.ds(15 * DBIG, DLAST)],
                            agg_out.at[c, pl.ds(15 * DBIG, DLAST)])
        if with_deg:
            pltpu.sync_copy(accd.at[pl.ds(s * DBIG, DBIG)],
                            deg_out.at[c, 0, pl.ds(s * DBIG, DBIG)])

    fn = pl.kernel(body, out_type=tuple(out_type), mesh=_mesh(),
                   scratch_types=scratch, compiler_params=_SC_PARAMS)
    return fn(h, e3)


def _seg_sum1(p, e3):
    scratch = [
        pltpu.VMEM_SHARED((NPAD,), jnp.float32),
        pltpu.VMEM((NPAD,), jnp.float32),
        pltpu.VMEM((SCH, CH), jnp.int32),
        pltpu.VMEM((SCH, CH), jnp.int32),
        pltpu.VMEM((DBIG,), jnp.float32),
        pltpu.VMEM((CH,), jnp.float32),
        pltpu.VMEM((CH,), jnp.float32),
        pltpu.SemaphoreType.DMA,
        pltpu.SemaphoreType.DMA,
    ]

    def body(p_hbm, e3_hbm, out, accd, table, sidx, didx,
             zdeg, valbuf0, valbuf1, sem0, sem1):
        c = lax.axis_index("c")
        s = lax.axis_index("s")
        w = c * 16 + s

        z16 = jnp.zeros((16,), jnp.float32)

        @pl.loop(0, DBIG // 16)
        def _(i):
            zdeg[pl.ds(i * 16, 16)] = z16
        pltpu.sync_copy(zdeg, accd.at[pl.ds(s * DBIG, DBIG)])
        pltpu.sync_copy(p_hbm, table)
        plsc.subcore_barrier()

        def fill(j, vb):
            for q in range(CH // 16):
                iv = sidx[j, pl.ds(q * 16, 16)]
                vb[pl.ds(q * 16, 16)] = plsc.load_gather(table, [iv])
            iv = sidx[j, pl.ds(CH - 16, 16)]
            vb[pl.ds(CH - 16, 16)] = plsc.load_gather(table, [iv])

        def sstart(j, vb, sem):
            pltpu.async_copy(vb, accd.at[didx.at[j]], sem, add=True)

        def swait(j, vb, sem):
            pltpu.make_async_copy(vb, accd.at[didx.at[j]], sem).wait()

        for g in range(0, NCH, SCH):
            pltpu.sync_copy(e3_hbm.at[0, w, pl.ds(g, SCH)], sidx)
            pltpu.sync_copy(e3_hbm.at[1, w, pl.ds(g, SCH)], didx)
            fill(0, valbuf0)
            sstart(0, valbuf0, sem0)

            @pl.loop(1, SCH - 1, step=2)
            def _(j):
                fill(j, valbuf1)
                sstart(j, valbuf1, sem1)
                swait(j - 1, valbuf0, sem0)
                fill(j + 1, valbuf0)
                sstart(j + 1, valbuf0, sem0)
                swait(j, valbuf1, sem1)

            fill(SCH - 1, valbuf1)
            sstart(SCH - 1, valbuf1, sem1)
            swait(SCH - 2, valbuf0, sem0)
            swait(SCH - 1, valbuf1, sem1)

        plsc.subcore_barrier()
        pltpu.sync_copy(accd.at[pl.ds(s * DBIG, DBIG)],
                        out.at[c, 0, pl.ds(s * DBIG, DBIG)])

    fn = pl.kernel(body,
                   out_type=jax.ShapeDtypeStruct((2, 1, NPAD), jnp.float32),
                   mesh=_mesh(), scratch_types=scratch,
                   compiler_params=_SC_PARAMS_GATHER)
    return fn(p, e3)


_RB1 = 2048
_PB = _RB1 // 128


def _mm(a, b):
    return jnp.dot(a, b, preferred_element_type=jnp.float32)


def _tc_layer0_body(aA, aB, dA, dB, h_, wn, ws, bb, o):
    invd = 1.0 / jnp.maximum(dA[...] + dB[...], 1.0)
    agg = (aA[0] + aB[0]) * invd
    out = _mm(agg, wn[...]) + _mm(h_[...], ws[...]) + bb[...]
    o[...] = jnp.maximum(out, 0.0)


def _tc_layer0(aggp, degA, degB, h, Wn, Ws, b2d):
    row = lambda i: (i, 0)
    fixed = lambda i: (0, 0)
    return pl.pallas_call(
        _tc_layer0_body,
        grid=(5,),
        in_specs=[
            pl.BlockSpec((1, _RB1, H), lambda i: (0, i, 0)),
            pl.BlockSpec((1, _RB1, H), lambda i: (1, i, 0)),
            pl.BlockSpec((_RB1, 1), row),
            pl.BlockSpec((_RB1, 1), row),
            pl.BlockSpec((_RB1, H), row),
            pl.BlockSpec((H, H), fixed),
            pl.BlockSpec((H, H), fixed),
            pl.BlockSpec((1, H), fixed),
        ],
        out_specs=pl.BlockSpec((_RB1, H), row),
        out_shape=jax.ShapeDtypeStruct((N, H), jnp.float32),
    )(aggp, aggp, degA, degB, h, Wn, Ws, b2d)


def _tc_layer1_body(aA, aB, dA, dB, h_, wn, ws, bb, wn2, ws2, b2, op, os):
    invd = 1.0 / jnp.maximum(dA[...] + dB[...], 1.0)
    agg = (aA[0] + aB[0]) * invd
    h2 = _mm(agg, wn[...]) + _mm(h_[...], ws[...]) + bb[...]
    h2 = jnp.maximum(h2, 0.0)
    op[...] = jnp.sum(h2 * wn2[...], axis=1).reshape(_PB, 128)
    os[...] = jnp.sum(h2 * ws2[...], axis=1).reshape(_PB, 128) + b2[...]


def _tc_layer1(aggp, degA, degB, h, Wn, Ws, b2d, wn2, ws2, b11):
    row = lambda i: (i, 0)
    fixed = lambda i: (0, 0)
    return pl.pallas_call(
        _tc_layer1_body,
        grid=(5,),
        in_specs=[
            pl.BlockSpec((1, _RB1, H), lambda i: (0, i, 0)),
            pl.BlockSpec((1, _RB1, H), lambda i: (1, i, 0)),
            pl.BlockSpec((_RB1, 1), row),
            pl.BlockSpec((_RB1, 1), row),
            pl.BlockSpec((_RB1, H), row),
            pl.BlockSpec((H, H), fixed),
            pl.BlockSpec((H, H), fixed),
            pl.BlockSpec((1, H), fixed),
            pl.BlockSpec((1, H), fixed),
            pl.BlockSpec((1, H), fixed),
            pl.BlockSpec((1, 1), fixed),
        ],
        out_specs=[
            pl.BlockSpec((_PB, 128), row),
            pl.BlockSpec((_PB, 128), row),
        ],
        out_shape=[
            jax.ShapeDtypeStruct((PR, 128), jnp.float32),
            jax.ShapeDtypeStruct((PR, 128), jnp.float32),
        ],
    )(aggp, aggp, degA, degB, h, Wn, Ws, b2d, wn2, ws2, b11)


def _tc_combine_body(aa, dd, s2, o):
    d = dd[0] + dd[1]
    invd = 1.0 / jnp.maximum(d, 1.0)
    o[...] = (aa[0] + aa[1]) * invd + s2[...]


def _tc_combine(a2pk, degpk, s2):
    return pl.pallas_call(
        _tc_combine_body,
        grid=(5,),
        in_specs=[
            pl.BlockSpec((2, PR // 5, 128), lambda i: (0, i, 0)),
            pl.BlockSpec((2, PR // 5, 128), lambda i: (0, i, 0)),
            pl.BlockSpec((PR // 5, 128), lambda i: (i, 0)),
        ],
        out_specs=pl.BlockSpec((PR // 5, 128), lambda i: (i, 0)),
        out_shape=jax.ShapeDtypeStruct((PR, 128), jnp.float32),
    )(a2pk, degpk, s2)


def kernel(x, edge_index, Wn0, Ws0, b0, Wn1, Ws1, b1, Wn2, Ws2, b2):
    e3 = edge_index.reshape(2, NW, NCH, CH)

    agg0, degp = _seg_sum(x, e3, with_deg=True)
    degpk = degp.reshape(2, PR, 128)
    degA = degp[0, 0, :N].reshape(N, 1)
    degB = degp[1, 0, :N].reshape(N, 1)

    h1 = _tc_layer0(agg0, degA, degB, x, Wn0, Ws0, b0.reshape(1, H))
    (agg1,) = _seg_sum(h1, e3, with_deg=False)
    p2, s2 = _tc_layer1(agg1, degA, degB, h1, Wn1, Ws1,
                        b1.reshape(1, H), Wn2.reshape(1, H),
                        Ws2.reshape(1, H), b2.reshape(1, 1))
    agg2p = _seg_sum1(p2.reshape(NPAD), e3)
    out = _tc_combine(agg2p.reshape(2, PR, 128), degpk, s2)
    return out.reshape(NPAD)[:N]
